# speedup vs baseline: 2.7446x; 2.7446x over previous
import functools

import jax
import jax.numpy as jnp
from jax import lax
from jax.experimental import pallas as pl
from jax.experimental.pallas import tpu as pltpu
from jax.experimental.pallas import tpu_sc as plsc

B, S, HID = 4, 8192, 128
N = B * S

NC, NS, L = 2, 16, 16
NW = NC * NS
NPW = N // NW
C = 128
NCHUNK = NPW // C

_mesh = plsc.VectorSubcoreMesh(core_axis_name="c", subcore_axis_name="s")


@functools.partial(
    pl.kernel,
    mesh=_mesh,
    out_type=jax.ShapeDtypeStruct((N, HID), jnp.float32),
    scratch_types=[
        pltpu.VMEM((NPW,), jnp.int32),
        pltpu.VMEM((NPW,), jnp.int32),
        pltpu.VMEM((C, HID), jnp.float32),
        pltpu.VMEM((C, HID), jnp.float32),
        pltpu.SemaphoreType.DMA,
        pltpu.SemaphoreType.DMA,
    ],
)
def _embed_add(wids_hbm, pids_hbm, wtab_hbm, ptab_hbm, out_hbm,
               widx_v, pidx_v, bufw_v, bufp_v, semw, semp):
    wid = lax.axis_index("s") * NC + lax.axis_index("c")
    base = wid * NPW
    pltpu.sync_copy(wids_hbm.at[pl.ds(base, NPW)], widx_v)
    pltpu.sync_copy(pids_hbm.at[pl.ds(base, NPW)], pidx_v)

    def chunk_body(c, carry):
        off = c * C
        cw = pltpu.async_copy(wtab_hbm.at[widx_v.at[pl.ds(off, C)]], bufw_v,
                              semw)
        cp = pltpu.async_copy(ptab_hbm.at[pidx_v.at[pl.ds(off, C)]], bufp_v,
                              semp)
        cw.wait()
        cp.wait()

        def add_row(i, carry2):
            for j in range(HID // L):
                sl = pl.ds(j * L, L)
                bufw_v[i, sl] = bufw_v[i, sl] + bufp_v[i, sl]
            return carry2

        lax.fori_loop(0, C, add_row, 0)
        pltpu.sync_copy(bufw_v, out_hbm.at[pl.ds(base + off, C)])
        return carry

    lax.fori_loop(0, NCHUNK, chunk_body, 0)


def kernel(input_ids, position_ids, word_embeddings, position_embeddings):
    wids = input_ids.reshape(-1).astype(jnp.int32)
    pids = position_ids.reshape(-1).astype(jnp.int32)
    out = _embed_add(wids, pids, word_embeddings, position_embeddings)
    return out.reshape(B, S, HID)


# speedup vs baseline: 3.3163x; 1.2083x over previous
import functools

import jax
import jax.numpy as jnp
from jax import lax
from jax.experimental import pallas as pl
from jax.experimental.pallas import tpu as pltpu
from jax.experimental.pallas import tpu_sc as plsc

B, S, HID = 4, 8192, 128
N = B * S

NC, NS, L = 2, 16, 16
NW = NC * NS
NPW = N // NW
C = 128
NCHUNK = NPW // C
NBW = 3
NBP = 2

_mesh = plsc.VectorSubcoreMesh(core_axis_name="c", subcore_axis_name="s")


def _add_chunk(w, p):
    @plsc.parallel_loop(0, C, unroll=4)
    def add_row(i):
        for j in range(HID // L):
            sl = pl.ds(j * L, L)
            w[i, sl] = w[i, sl] + p[i, sl]


@functools.partial(
    pl.kernel,
    mesh=_mesh,
    out_type=jax.ShapeDtypeStruct((N, HID), jnp.float32),
    scratch_types=(
        [pltpu.VMEM((NPW,), jnp.int32)] * 2
        + [pltpu.VMEM((C, HID), jnp.float32)] * (NBW + NBP)
        + [pltpu.SemaphoreType.DMA] * (NBW + NBP + 1)
    ),
)
def _embed_add(wids_hbm, pids_hbm, wtab_hbm, ptab_hbm, out_hbm,
               widx_v, pidx_v, bw0, bw1, bw2, bp0, bp1,
               sw0, sw1, sw2, sp0, sp1, so):
    bufw, bufp = (bw0, bw1, bw2), (bp0, bp1)
    semw, semp = (sw0, sw1, sw2), (sp0, sp1)
    wid = lax.axis_index("s") * NC + lax.axis_index("c")
    base = wid * NPW
    pltpu.sync_copy(wids_hbm.at[pl.ds(base, NPW)], widx_v)
    pltpu.sync_copy(pids_hbm.at[pl.ds(base, NPW)], pidx_v)

    def fire(c):
        off = c * C
        gw = pltpu.async_copy(wtab_hbm.at[widx_v.at[pl.ds(off, C)]],
                              bufw[c % NBW], semw[c % NBW])
        gp = pltpu.async_copy(ptab_hbm.at[pidx_v.at[pl.ds(off, C)]],
                              bufp[c % NBP], semp[c % NBP])
        return gw, gp

    pend = fire(0)
    outs = [None] * NCHUNK
    for c in range(NCHUNK):
        if c + 1 < NCHUNK:
            if c + 1 >= NBW:
                outs[c + 1 - NBW].wait()
            nxt = fire(c + 1)
        pend[0].wait()
        pend[1].wait()
        w, p = bufw[c % NBW], bufp[c % NBP]
        _add_chunk(w, p)
        outs[c] = pltpu.async_copy(w, out_hbm.at[pl.ds(base + c * C, C)], so)
        if c + 1 < NCHUNK:
            pend = nxt
    for c in range(max(0, NCHUNK - NBW), NCHUNK):
        outs[c].wait()


def kernel(input_ids, position_ids, word_embeddings, position_embeddings):
    wids = input_ids.reshape(-1).astype(jnp.int32)
    pids = position_ids.reshape(-1).astype(jnp.int32)
    out = _embed_add(wids, pids, word_embeddings, position_embeddings)
    return out.reshape(B, S, HID)


# speedup vs baseline: 3.3511x; 1.0105x over previous
import functools

import jax
import jax.numpy as jnp
from jax import lax
from jax.experimental import pallas as pl
from jax.experimental.pallas import tpu as pltpu
from jax.experimental.pallas import tpu_sc as plsc

B, S, HID = 4, 8192, 128
N = B * S

NC, NS, L = 2, 16, 16
NW = NC * NS
NPW = N // NW
C = 128
NCHUNK = NPW // C
NBW = 3
NBP = 2

_mesh = plsc.VectorSubcoreMesh(core_axis_name="c", subcore_axis_name="s")


def _add_chunk(w, p):
    @plsc.parallel_loop(0, C, unroll=4)
    def add_row(i):
        for j in range(HID // L):
            sl = pl.ds(j * L, L)
            plsc.addupdate(w.at[i, sl], p[i, sl])


@functools.partial(
    pl.kernel,
    mesh=_mesh,
    out_type=jax.ShapeDtypeStruct((N, HID), jnp.float32),
    scratch_types=(
        [pltpu.VMEM((NPW,), jnp.int32)] * 2
        + [pltpu.VMEM((C, HID), jnp.float32)] * (NBW + NBP)
        + [pltpu.SemaphoreType.DMA] * (NBW + NBP + 1)
    ),
)
def _embed_add(wids_hbm, pids_hbm, wtab_hbm, ptab_hbm, out_hbm,
               widx_v, pidx_v, bw0, bw1, bw2, bp0, bp1,
               sw0, sw1, sw2, sp0, sp1, so):
    bufw, bufp = (bw0, bw1, bw2), (bp0, bp1)
    semw, semp = (sw0, sw1, sw2), (sp0, sp1)
    wid = lax.axis_index("s") * NC + lax.axis_index("c")
    base = wid * NPW
    pltpu.sync_copy(wids_hbm.at[pl.ds(base, NPW)], widx_v)
    pltpu.sync_copy(pids_hbm.at[pl.ds(base, NPW)], pidx_v)

    def fire(c):
        off = c * C
        gw = pltpu.async_copy(wtab_hbm.at[widx_v.at[pl.ds(off, C)]],
                              bufw[c % NBW], semw[c % NBW])
        gp = pltpu.async_copy(ptab_hbm.at[pidx_v.at[pl.ds(off, C)]],
                              bufp[c % NBP], semp[c % NBP])
        return gw, gp

    pend = fire(0)
    outs = [None] * NCHUNK
    for c in range(NCHUNK):
        if c + 1 < NCHUNK:
            if c + 1 >= NBW:
                outs[c + 1 - NBW].wait()
            nxt = fire(c + 1)
        pend[0].wait()
        pend[1].wait()
        w, p = bufw[c % NBW], bufp[c % NBP]
        _add_chunk(w, p)
        outs[c] = pltpu.async_copy(w, out_hbm.at[pl.ds(base + c * C, C)], so)
        if c + 1 < NCHUNK:
            pend = nxt
    for c in range(max(0, NCHUNK - NBW), NCHUNK):
        outs[c].wait()


def kernel(input_ids, position_ids, word_embeddings, position_embeddings):
    wids = input_ids.reshape(-1).astype(jnp.int32)
    pids = position_ids.reshape(-1).astype(jnp.int32)
    out = _embed_add(wids, pids, word_embeddings, position_embeddings)
    return out.reshape(B, S, HID)


# speedup vs baseline: 3.4758x; 1.0372x over previous
import functools

import jax
import jax.numpy as jnp
from jax import lax
from jax.experimental import pallas as pl
from jax.experimental.pallas import tpu as pltpu
from jax.experimental.pallas import tpu_sc as plsc

B, S, HID = 4, 8192, 128
N = B * S

NC, NS, L = 2, 16, 16
NW = NC * NS
NPW = N // NW
C = 128
NCHUNK = NPW // C
NB = 4

_mesh = plsc.VectorSubcoreMesh(core_axis_name="c", subcore_axis_name="s")


@functools.partial(
    pl.kernel,
    mesh=_mesh,
    out_type=jax.ShapeDtypeStruct((N, HID), jnp.float32),
    scratch_types=(
        [pltpu.VMEM((NPW,), jnp.int32)] * 2
        + [pltpu.VMEM((C, HID), jnp.float32)] * NB
        + [pltpu.SemaphoreType.DMA] * (2 * NB + 1)
    ),
)
def _embed_add(wids_hbm, pids_hbm, wtab_hbm, ptab_hbm, out_hbm,
               widx_v, pidx_v, b0, b1, b2, b3,
               sw0, sw1, sw2, sw3, sa0, sa1, sa2, sa3, so):
    bufs = (b0, b1, b2, b3)
    semw = (sw0, sw1, sw2, sw3)
    sema = (sa0, sa1, sa2, sa3)
    wid = lax.axis_index("s") * NC + lax.axis_index("c")
    base = wid * NPW
    pltpu.sync_copy(wids_hbm.at[pl.ds(base, NPW)], widx_v)
    pltpu.sync_copy(pids_hbm.at[pl.ds(base, NPW)], pidx_v)

    def fire_w(c):
        return pltpu.async_copy(wtab_hbm.at[widx_v.at[pl.ds(c * C, C)]],
                                bufs[c % NB], semw[c % NB])

    def fire_p(c):
        return pltpu.async_copy(ptab_hbm.at[pidx_v.at[pl.ds(c * C, C)]],
                                bufs[c % NB], sema[c % NB], add=True)

    def fire_out(c):
        return pltpu.async_copy(bufs[c % NB],
                                out_hbm.at[pl.ds(base + c * C, C)], so)

    gw = [None] * NCHUNK
    gp = [None] * NCHUNK
    outs = [None] * NCHUNK
    gw[0] = fire_w(0)
    if NCHUNK > 1:
        gw[1] = fire_w(1)
    for c in range(NCHUNK):
        gw[c].wait()
        gp[c] = fire_p(c)
        if c + 2 < NCHUNK:
            if c + 2 >= NB:
                outs[c + 2 - NB].wait()
            gw[c + 2] = fire_w(c + 2)
        gp[c].wait()
        outs[c] = fire_out(c)
    for c in range(max(0, NCHUNK - NB), NCHUNK):
        outs[c].wait()


def kernel(input_ids, position_ids, word_embeddings, position_embeddings):
    wids = input_ids.reshape(-1).astype(jnp.int32)
    pids = position_ids.reshape(-1).astype(jnp.int32)
    out = _embed_add(wids, pids, word_embeddings, position_embeddings)
    return out.reshape(B, S, HID)


# speedup vs baseline: 3.5404x; 1.0186x over previous
import functools

import jax
import jax.numpy as jnp
from jax import lax
from jax.experimental import pallas as pl
from jax.experimental.pallas import tpu as pltpu
from jax.experimental.pallas import tpu_sc as plsc

B, S, HID = 4, 8192, 128
N = B * S

NC, NS, L = 2, 16, 16
NW = NC * NS
NPW = N // NW
WPB = S // NPW
C = 128
NCHUNK = NPW // C
NB = 4

_mesh = plsc.VectorSubcoreMesh(core_axis_name="c", subcore_axis_name="s")


@functools.partial(
    pl.kernel,
    mesh=_mesh,
    out_type=jax.ShapeDtypeStruct((B, S, HID), jnp.float32),
    scratch_types=(
        [pltpu.VMEM((NPW,), jnp.int32)] * 2
        + [pltpu.VMEM((C, HID), jnp.float32)] * NB
        + [pltpu.SemaphoreType.DMA] * (2 * NB + 1)
    ),
)
def _embed_add(wids_hbm, pids_hbm, wtab_hbm, ptab_hbm, out_hbm,
               widx_v, pidx_v, b0, b1, b2, b3,
               sw0, sw1, sw2, sw3, sa0, sa1, sa2, sa3, so):
    bufs = (b0, b1, b2, b3)
    semw = (sw0, sw1, sw2, sw3)
    sema = (sa0, sa1, sa2, sa3)
    wid = lax.axis_index("s") * NC + lax.axis_index("c")
    row = wid // WPB
    off = (wid % WPB) * NPW
    pltpu.sync_copy(wids_hbm.at[row, pl.ds(off, NPW)], widx_v)
    pltpu.sync_copy(pids_hbm.at[row, pl.ds(off, NPW)], pidx_v)

    def fire_w(c):
        return pltpu.async_copy(wtab_hbm.at[widx_v.at[pl.ds(c * C, C)]],
                                bufs[c % NB], semw[c % NB])

    def fire_p(c):
        return pltpu.async_copy(ptab_hbm.at[pidx_v.at[pl.ds(c * C, C)]],
                                bufs[c % NB], sema[c % NB], add=True)

    def fire_out(c):
        return pltpu.async_copy(bufs[c % NB],
                                out_hbm.at[row, pl.ds(off + c * C, C)], so)

    gw = [None] * NCHUNK
    gp = [None] * NCHUNK
    outs = [None] * NCHUNK
    gw[0] = fire_w(0)
    if NCHUNK > 1:
        gw[1] = fire_w(1)
    for c in range(NCHUNK):
        gw[c].wait()
        gp[c] = fire_p(c)
        if c + 2 < NCHUNK:
            if c + 2 >= NB:
                outs[c + 2 - NB].wait()
            gw[c + 2] = fire_w(c + 2)
        gp[c].wait()
        outs[c] = fire_out(c)
    for c in range(max(0, NCHUNK - NB), NCHUNK):
        outs[c].wait()


def kernel(input_ids, position_ids, word_embeddings, position_embeddings):
    return _embed_add(input_ids.astype(jnp.int32),
                      position_ids.astype(jnp.int32),
                      word_embeddings, position_embeddings)
